# Initial kernel scaffold; baseline (speedup 1.0000x reference)
import jax
import jax.numpy as jnp
from jax.experimental import pallas as pl


def kernel(node_features, rigids, edge_features, edge_index, W_gate, b_gate, W_bias, b_bias, W_qk, b_qk, W_v, b_v, W_out, b_out, eps):
    raise NotImplementedError("write your pallas kernel here")


# speedup vs baseline: 1.1735x; 1.1735x over previous
import functools

import jax
import jax.numpy as jnp
import numpy as np
from jax.experimental import pallas as pl

N = 384
K = 60
SK = 20
C_S = 384
C_Z = 128
H = 4
NUM_RBF = 64
DH = C_Z // H
E = N * K

BN = 8
BNK = BN * K
BSK = BN * SK

_D_MAX = 20.0
_MU_STEP = _D_MAX / (NUM_RBF - 1)
_INV_SIGMA = NUM_RBF / _D_MAX
_SCALE = 1.0 / float(np.sqrt(C_Z))


@functools.lru_cache(maxsize=1)
def _perm_flat_const():
    with jax.ensure_compile_time_eval():
        u = jax.random.uniform(jax.random.key(42), (N, K, K))
        idx = jnp.argsort(u, axis=-1)[..., :SK]
    idx_np = np.asarray(jax.device_get(idx), dtype=np.int32)
    flat = np.arange(N, dtype=np.int32)[:, None, None] * K + idx_np
    return flat.reshape(-1)


def _gate_proj_kernel(nf_ref, w_ref, out_ref):
    out_ref[...] = jnp.dot(nf_ref[...], w_ref[...],
                           preferred_element_type=jnp.float32)


def _main_kernel(edges_ref, dist_ref, gsum_ref,
                 wq_ref, wk_ref, wv_ref, wo_ref,
                 bq_ref, bk_ref, bv_ref, bo_ref,
                 wbt_ref, bb_ref, out_ref):
    e = edges_ref[...]
    q = jnp.dot(e, wq_ref[...],
                preferred_element_type=jnp.float32) + bq_ref[...]
    e_sub = e.reshape(BN, K, C_Z)[:, :SK, :].reshape(BSK, C_Z)
    ks = jnp.dot(e_sub, wk_ref[...],
                 preferred_element_type=jnp.float32) + bk_ref[...]
    vs = jnp.dot(e_sub, wv_ref[...],
                 preferred_element_type=jnp.float32) + bv_ref[...]

    dist = dist_ref[...]
    mu = jax.lax.broadcasted_iota(jnp.int32, (BNK, SK, NUM_RBF), 2)\
        .astype(jnp.float32) * _MU_STEP
    arg = (dist[:, :, None] - mu) * _INV_SIGMA
    phi = jnp.exp(-(arg * arg))

    row_bn = jax.lax.broadcasted_iota(jnp.int32, (BNK, BSK), 0) // K
    col_bn = jax.lax.broadcasted_iota(jnp.int32, (BNK, BSK), 1) // SK
    mask = row_bn == col_bn

    u_parts = []
    for h in range(H):
        wb = wbt_ref[h : h + 1, :].reshape(1, 1, NUM_RBF)
        raw = jnp.sum(phi * wb, axis=-1) + bb_ref[0, h]
        gate = jax.nn.sigmoid(gsum_ref[h])
        bias = gate * raw
        qh = q[:, h * DH : (h + 1) * DH]
        kh = ks[:, h * DH : (h + 1) * DH]
        s = jax.lax.dot_general(qh, kh, (((1,), (1,)), ((), ())),
                                preferred_element_type=jnp.float32) * _SCALE
        bias_big = jnp.concatenate([bias] * BN, axis=1)
        s = jnp.where(mask, s + bias_big, -1e30)
        m = jnp.max(s, axis=1, keepdims=True)
        p = jnp.exp(s - m)
        p = p / jnp.sum(p, axis=1, keepdims=True)
        vh = vs[:, h * DH : (h + 1) * DH]
        u_parts.append(jnp.dot(p, vh, preferred_element_type=jnp.float32))
    u = jnp.concatenate(u_parts, axis=1)
    out_ref[...] = jnp.dot(u, wo_ref[...],
                           preferred_element_type=jnp.float32) + bo_ref[...]


def kernel(node_features, rigids, edge_features, edge_index,
           W_gate, b_gate, W_bias, b_bias, W_qk, b_qk, W_v, b_v,
           W_out, b_out, eps):
    dst = edge_index[0].reshape(N, K)
    perm_flat = jnp.asarray(_perm_flat_const())
    ge2 = jnp.take(dst.reshape(-1), perm_flat).reshape(N, K, SK)

    w_cat = jnp.concatenate([W_gate[:C_S], W_gate[C_S:]], axis=1)
    g = pl.pallas_call(
        _gate_proj_kernel,
        out_shape=jax.ShapeDtypeStruct((N, 2 * H), jnp.float32),
    )(node_features, w_cat)

    dst_flat = dst.reshape(-1)
    ge2_flat = ge2.reshape(-1)
    ra = jnp.take(rigids, dst_flat, axis=0).reshape(N * K, 1, 3)
    rb = jnp.take(rigids, ge2_flat, axis=0).reshape(N * K, SK, 3)
    diff = ra - rb + eps
    dist = jnp.sqrt(jnp.sum(diff * diff, axis=-1))

    g1 = jnp.take(g[:, :H], dst_flat, axis=0).reshape(N * K, 1, H)
    g2 = jnp.take(g[:, H:], ge2_flat, axis=0).reshape(N * K, SK, H)
    gsum = g1 + g2 + b_gate
    gsum_t = jnp.transpose(gsum, (2, 0, 1))

    wq = W_qk[:, :C_Z]
    wk = W_qk[:, C_Z:]
    bq = b_qk[:C_Z].reshape(1, C_Z)
    bk = b_qk[C_Z:].reshape(1, C_Z)
    bv = b_v.reshape(1, C_Z)
    bo = b_out.reshape(1, C_Z)
    wbt = W_bias.T
    bb = b_bias.reshape(1, H)

    grid = (N // BN,)
    out = pl.pallas_call(
        _main_kernel,
        grid=grid,
        in_specs=[
            pl.BlockSpec((BNK, C_Z), lambda i: (i, 0)),
            pl.BlockSpec((BNK, SK), lambda i: (i, 0)),
            pl.BlockSpec((H, BNK, SK), lambda i: (0, i, 0)),
            pl.BlockSpec((C_Z, C_Z), lambda i: (0, 0)),
            pl.BlockSpec((C_Z, C_Z), lambda i: (0, 0)),
            pl.BlockSpec((C_Z, C_Z), lambda i: (0, 0)),
            pl.BlockSpec((C_Z, C_Z), lambda i: (0, 0)),
            pl.BlockSpec((1, C_Z), lambda i: (0, 0)),
            pl.BlockSpec((1, C_Z), lambda i: (0, 0)),
            pl.BlockSpec((1, C_Z), lambda i: (0, 0)),
            pl.BlockSpec((1, C_Z), lambda i: (0, 0)),
            pl.BlockSpec((H, NUM_RBF), lambda i: (0, 0)),
            pl.BlockSpec((1, H), lambda i: (0, 0)),
        ],
        out_specs=pl.BlockSpec((BNK, C_Z), lambda i: (i, 0)),
        out_shape=jax.ShapeDtypeStruct((E, C_Z), jnp.float32),
    )(edge_features, dist, gsum_t, wq, wk, W_v, W_out,
      bq, bk, bv, bo, wbt, bb)
    return out


# speedup vs baseline: 5.7853x; 4.9299x over previous
import functools

import jax
import jax.numpy as jnp
import numpy as np
from jax import lax
from jax.experimental import pallas as pl
from jax.experimental.pallas import tpu as pltpu
from jax.experimental.pallas import tpu_sc as plsc

N = 384
K = 60
SK = 20
C_S = 384
C_Z = 128
H = 4
NUM_RBF = 64
DH = C_Z // H
E = N * K

BN = 8
BNK = BN * K
BSK = BN * SK

_D_MAX = 20.0
_MU_STEP = _D_MAX / (NUM_RBF - 1)
_INV_SIGMA = NUM_RBF / _D_MAX
_SCALE = 1.0 / float(np.sqrt(C_Z))


def _rotl32(x, d):
    return ((x << np.uint32(d)) | (x >> np.uint32(32 - d))).astype(np.uint32)


def _threefry2x32(k0, k1, x0, x1):
    x0 = x0.astype(np.uint32).copy()
    x1 = x1.astype(np.uint32).copy()
    k0 = np.uint32(k0)
    k1 = np.uint32(k1)
    k2 = np.uint32(k0 ^ k1 ^ np.uint32(0x1BD11BDA))
    rot = [(13, 15, 26, 6), (17, 29, 16, 24)]
    ks = [k0, k1, k2]
    x0 = (x0 + k0).astype(np.uint32)
    x1 = (x1 + k1).astype(np.uint32)
    for r in range(5):
        for d in rot[r % 2]:
            x0 = (x0 + x1).astype(np.uint32)
            x1 = _rotl32(x1, d)
            x1 = (x1 ^ x0).astype(np.uint32)
        x0 = (x0 + ks[(r + 1) % 3]).astype(np.uint32)
        x1 = (x1 + ks[(r + 2) % 3] + np.uint32(r + 1)).astype(np.uint32)
    return x0, x1


@functools.lru_cache(maxsize=1)
def _perm_flat_const():
    n = N * K * K
    b0, b1 = _threefry2x32(0, 42, np.zeros(n, np.uint32),
                           np.arange(n, dtype=np.uint32))
    bits = (b0 ^ b1).astype(np.uint32)
    fl = ((bits >> np.uint32(9)) | np.uint32(0x3F800000)).view(np.float32) \
        - np.float32(1.0)
    u = fl.reshape(N, K, K)
    idx_np = np.argsort(u, axis=-1, kind="stable")[..., :SK].astype(np.int32)
    flat = np.arange(N, dtype=np.int32)[:, None, None] * K + idx_np
    return flat.reshape(-1)


_TD = 16
_NW = 32


def _sc_gather(table, idx, rows, chunk):
    b_per_w = rows // _NW
    n_chunks = b_per_w // chunk
    mesh = plsc.VectorSubcoreMesh(core_axis_name="c", subcore_axis_name="s")

    @functools.partial(
        pl.kernel, mesh=mesh,
        out_type=jax.ShapeDtypeStruct((rows, _TD), jnp.float32),
        compiler_params=pltpu.CompilerParams(use_tc_tiling_on_sc=False),
        scratch_types=[
            pltpu.VMEM((chunk,), jnp.int32),
            pltpu.VMEM((chunk, _TD), jnp.float32),
            pltpu.SemaphoreType.DMA,
        ],
    )
    def k(table_hbm, idx_hbm, out_hbm, idx_v, rows_v, sem):
        wid = lax.axis_index("s") * 2 + lax.axis_index("c")
        base = wid * b_per_w

        def body(c, _):
            off = base + c * chunk
            pltpu.sync_copy(idx_hbm.at[pl.ds(off, chunk)], idx_v)
            pltpu.async_copy(table_hbm.at[idx_v], rows_v, sem).wait()
            pltpu.sync_copy(rows_v, out_hbm.at[pl.ds(off, chunk)])
            return ()

        lax.fori_loop(0, n_chunks, body, ())

    return k(table, idx)


def _gate_proj_kernel(nf_ref, w_ref, out_ref):
    out_ref[...] = jnp.dot(nf_ref[...], w_ref[...],
                           preferred_element_type=jnp.float32)


def _main_kernel(edges_ref, dist_ref, gsum_ref,
                 wq_ref, wk_ref, wv_ref, wo_ref,
                 bq_ref, bk_ref, bv_ref, bo_ref,
                 wbt_ref, bb_ref, out_ref):
    e = edges_ref[...]
    q = jnp.dot(e, wq_ref[...],
                preferred_element_type=jnp.float32) + bq_ref[...]
    e_sub = e.reshape(BN, K, C_Z)[:, :SK, :].reshape(BSK, C_Z)
    ks = jnp.dot(e_sub, wk_ref[...],
                 preferred_element_type=jnp.float32) + bk_ref[...]
    vs = jnp.dot(e_sub, wv_ref[...],
                 preferred_element_type=jnp.float32) + bv_ref[...]

    dist = dist_ref[...]
    mu = jax.lax.broadcasted_iota(jnp.int32, (BNK, SK, NUM_RBF), 2)\
        .astype(jnp.float32) * _MU_STEP
    arg = (dist[:, :, None] - mu) * _INV_SIGMA
    phi = jnp.exp(-(arg * arg))

    row_bn = jax.lax.broadcasted_iota(jnp.int32, (BNK, BSK), 0) // K
    col_bn = jax.lax.broadcasted_iota(jnp.int32, (BNK, BSK), 1) // SK
    mask = row_bn == col_bn

    u_parts = []
    for h in range(H):
        wb = wbt_ref[h : h + 1, :].reshape(1, 1, NUM_RBF)
        raw = jnp.sum(phi * wb, axis=-1) + bb_ref[0, h]
        gate = jax.nn.sigmoid(gsum_ref[h])
        bias = gate * raw
        qh = q[:, h * DH : (h + 1) * DH]
        kh = ks[:, h * DH : (h + 1) * DH]
        s = jax.lax.dot_general(qh, kh, (((1,), (1,)), ((), ())),
                                preferred_element_type=jnp.float32) * _SCALE
        bias_big = jnp.concatenate([bias] * BN, axis=1)
        s = jnp.where(mask, s + bias_big, -1e30)
        m = jnp.max(s, axis=1, keepdims=True)
        p = jnp.exp(s - m)
        p = p / jnp.sum(p, axis=1, keepdims=True)
        vh = vs[:, h * DH : (h + 1) * DH]
        u_parts.append(jnp.dot(p, vh, preferred_element_type=jnp.float32))
    u = jnp.concatenate(u_parts, axis=1)
    out_ref[...] = jnp.dot(u, wo_ref[...],
                           preferred_element_type=jnp.float32) + bo_ref[...]


def kernel(node_features, rigids, edge_features, edge_index,
           W_gate, b_gate, W_bias, b_bias, W_qk, b_qk, W_v, b_v,
           W_out, b_out, eps):
    dst = edge_index[0].reshape(N, K)
    perm_flat = jnp.asarray(_perm_flat_const())

    w_cat = jnp.concatenate([W_gate[:C_S], W_gate[C_S:]], axis=1)
    g = pl.pallas_call(
        _gate_proj_kernel,
        out_shape=jax.ShapeDtypeStruct((N, 2 * H), jnp.float32),
    )(node_features, w_cat)

    table = jnp.concatenate(
        [rigids, g, jnp.zeros((N, _TD - 3 - 2 * H), jnp.float32)], axis=1)
    a_rows = _sc_gather(table, dst.reshape(-1).astype(jnp.int32), E, 720)
    b_rows = _sc_gather(a_rows, perm_flat, E * SK, 1800)

    ra = a_rows[:, 0:3].reshape(N * K, 1, 3)
    rb = b_rows[:, 0:3].reshape(N * K, SK, 3)
    diff = ra - rb + eps
    dist = jnp.sqrt(jnp.sum(diff * diff, axis=-1))

    g1 = a_rows[:, 3 : 3 + H].reshape(N * K, 1, H)
    g2 = b_rows[:, 3 + H : 3 + 2 * H].reshape(N * K, SK, H)
    gsum = g1 + g2 + b_gate
    gsum_t = jnp.transpose(gsum, (2, 0, 1))

    wq = W_qk[:, :C_Z]
    wk = W_qk[:, C_Z:]
    bq = b_qk[:C_Z].reshape(1, C_Z)
    bk = b_qk[C_Z:].reshape(1, C_Z)
    bv = b_v.reshape(1, C_Z)
    bo = b_out.reshape(1, C_Z)
    wbt = W_bias.T
    bb = b_bias.reshape(1, H)

    grid = (N // BN,)
    out = pl.pallas_call(
        _main_kernel,
        grid=grid,
        in_specs=[
            pl.BlockSpec((BNK, C_Z), lambda i: (i, 0)),
            pl.BlockSpec((BNK, SK), lambda i: (i, 0)),
            pl.BlockSpec((H, BNK, SK), lambda i: (0, i, 0)),
            pl.BlockSpec((C_Z, C_Z), lambda i: (0, 0)),
            pl.BlockSpec((C_Z, C_Z), lambda i: (0, 0)),
            pl.BlockSpec((C_Z, C_Z), lambda i: (0, 0)),
            pl.BlockSpec((C_Z, C_Z), lambda i: (0, 0)),
            pl.BlockSpec((1, C_Z), lambda i: (0, 0)),
            pl.BlockSpec((1, C_Z), lambda i: (0, 0)),
            pl.BlockSpec((1, C_Z), lambda i: (0, 0)),
            pl.BlockSpec((1, C_Z), lambda i: (0, 0)),
            pl.BlockSpec((H, NUM_RBF), lambda i: (0, 0)),
            pl.BlockSpec((1, H), lambda i: (0, 0)),
        ],
        out_specs=pl.BlockSpec((BNK, C_Z), lambda i: (i, 0)),
        out_shape=jax.ShapeDtypeStruct((E, C_Z), jnp.float32),
    )(edge_features, dist, gsum_t, wq, wk, W_v, W_out,
      bq, bk, bv, bo, wbt, bb)
    return out


# speedup vs baseline: 7.4037x; 1.2797x over previous
import functools

import jax
import jax.numpy as jnp
import numpy as np
from jax import lax
from jax.experimental import pallas as pl
from jax.experimental.pallas import tpu as pltpu
from jax.experimental.pallas import tpu_sc as plsc

N = 384
K = 60
SK = 20
C_S = 384
C_Z = 128
H = 4
NUM_RBF = 64
DH = C_Z // H
E = N * K

BN = 8
BNK = BN * K
BSK = BN * SK

_D_MAX = 20.0
_MU_STEP = _D_MAX / (NUM_RBF - 1)
_INV_SIGMA = NUM_RBF / _D_MAX
_SCALE = 1.0 / float(np.sqrt(C_Z))


def _rotl32(x, d):
    return ((x << np.uint32(d)) | (x >> np.uint32(32 - d))).astype(np.uint32)


def _threefry2x32(k0, k1, x0, x1):
    x0 = x0.astype(np.uint32).copy()
    x1 = x1.astype(np.uint32).copy()
    k0 = np.uint32(k0)
    k1 = np.uint32(k1)
    k2 = np.uint32(k0 ^ k1 ^ np.uint32(0x1BD11BDA))
    rot = [(13, 15, 26, 6), (17, 29, 16, 24)]
    ks = [k0, k1, k2]
    x0 = (x0 + k0).astype(np.uint32)
    x1 = (x1 + k1).astype(np.uint32)
    for r in range(5):
        for d in rot[r % 2]:
            x0 = (x0 + x1).astype(np.uint32)
            x1 = _rotl32(x1, d)
            x1 = (x1 ^ x0).astype(np.uint32)
        x0 = (x0 + ks[(r + 1) % 3]).astype(np.uint32)
        x1 = (x1 + ks[(r + 2) % 3] + np.uint32(r + 1)).astype(np.uint32)
    return x0, x1


@functools.lru_cache(maxsize=1)
def _perm_flat_const():
    n = N * K * K
    b0, b1 = _threefry2x32(0, 42, np.zeros(n, np.uint32),
                           np.arange(n, dtype=np.uint32))
    bits = (b0 ^ b1).astype(np.uint32)
    fl = ((bits >> np.uint32(9)) | np.uint32(0x3F800000)).view(np.float32) \
        - np.float32(1.0)
    u = fl.reshape(N, K, K)
    idx_np = np.argsort(u, axis=-1, kind="stable")[..., :SK].astype(np.int32)
    flat = np.arange(N, dtype=np.int32)[:, None, None] * K + idx_np
    return flat.reshape(-1)


_TD = 16
_TDB = 8
_NW = 32


def _sc_gather(table, idx, rows, width, chunk):
    b_per_w = rows // _NW
    n_chunks = b_per_w // chunk
    mesh = plsc.VectorSubcoreMesh(core_axis_name="c", subcore_axis_name="s")

    @functools.partial(
        pl.kernel, mesh=mesh,
        out_type=jax.ShapeDtypeStruct((rows, width), jnp.float32),
        compiler_params=pltpu.CompilerParams(use_tc_tiling_on_sc=False),
        scratch_types=[
            pltpu.VMEM((chunk,), jnp.int32),
            pltpu.VMEM((chunk, width), jnp.float32),
            pltpu.SemaphoreType.DMA,
        ],
    )
    def k(table_hbm, idx_hbm, out_hbm, idx_v, rows_v, sem):
        wid = lax.axis_index("s") * 2 + lax.axis_index("c")
        base = wid * b_per_w

        def body(c, _):
            off = base + c * chunk
            pltpu.sync_copy(idx_hbm.at[pl.ds(off, chunk)], idx_v)
            pltpu.async_copy(table_hbm.at[idx_v], rows_v, sem).wait()
            pltpu.sync_copy(rows_v, out_hbm.at[pl.ds(off, chunk)])
            return ()

        lax.fori_loop(0, n_chunks, body, ())

    return k(table, idx)


_BR = 15360


def _bias_kernel(dist_ref, gsum_ref, wbt_ref, out_ref):
    d = dist_ref[...]
    mu = jax.lax.broadcasted_iota(jnp.int32, (NUM_RBF, _BR), 0)\
        .astype(jnp.float32) * _MU_STEP
    arg = (d - mu) * _INV_SIGMA
    phi = jnp.exp(-(arg * arg))
    ones = jnp.ones((1, _BR), jnp.float32)
    phi_ext = jnp.concatenate([phi, ones], axis=0)
    raw = jnp.dot(wbt_ref[...], phi_ext,
                  preferred_element_type=jnp.float32)
    out_ref[...] = jax.nn.sigmoid(gsum_ref[...]) * raw


def _gate_proj_kernel(nf_ref, w_ref, out_ref):
    out_ref[...] = jnp.dot(nf_ref[...], w_ref[...],
                           preferred_element_type=jnp.float32)


def _main_kernel(edges_ref, bias_ref,
                 wq_ref, wk_ref, wv_ref, wo_ref,
                 bq_ref, bk_ref, bv_ref, bo_ref, out_ref):
    e = edges_ref[...]
    q = jnp.dot(e, wq_ref[...],
                preferred_element_type=jnp.float32) + bq_ref[...]
    e_sub = e.reshape(BN, K, C_Z)[:, :SK, :].reshape(BSK, C_Z)
    ks = jnp.dot(e_sub, wk_ref[...],
                 preferred_element_type=jnp.float32) + bk_ref[...]
    vs = jnp.dot(e_sub, wv_ref[...],
                 preferred_element_type=jnp.float32) + bv_ref[...]

    row_bn = jax.lax.broadcasted_iota(jnp.int32, (BNK, BSK), 0) // K
    col_bn = jax.lax.broadcasted_iota(jnp.int32, (BNK, BSK), 1) // SK
    mask = row_bn == col_bn

    u_parts = []
    for h in range(H):
        bias = bias_ref[h]
        qh = q[:, h * DH : (h + 1) * DH]
        kh = ks[:, h * DH : (h + 1) * DH]
        s = jax.lax.dot_general(qh, kh, (((1,), (1,)), ((), ())),
                                preferred_element_type=jnp.float32) * _SCALE
        bias_big = jnp.concatenate([bias] * BN, axis=1)
        s = jnp.where(mask, s + bias_big, -1e30)
        m = jnp.max(s, axis=1, keepdims=True)
        p = jnp.exp(s - m)
        p = p / jnp.sum(p, axis=1, keepdims=True)
        vh = vs[:, h * DH : (h + 1) * DH]
        u_parts.append(jnp.dot(p, vh, preferred_element_type=jnp.float32))
    u = jnp.concatenate(u_parts, axis=1)
    out_ref[...] = jnp.dot(u, wo_ref[...],
                           preferred_element_type=jnp.float32) + bo_ref[...]


def kernel(node_features, rigids, edge_features, edge_index,
           W_gate, b_gate, W_bias, b_bias, W_qk, b_qk, W_v, b_v,
           W_out, b_out, eps):
    dst = edge_index[0].reshape(N, K)
    perm_flat = jnp.asarray(_perm_flat_const())

    w_cat = jnp.concatenate([W_gate[:C_S], W_gate[C_S:]], axis=1)
    g = pl.pallas_call(
        _gate_proj_kernel,
        out_shape=jax.ShapeDtypeStruct((N, 2 * H), jnp.float32),
    )(node_features, w_cat)

    table = jnp.concatenate(
        [rigids, g, jnp.zeros((N, _TD - 3 - 2 * H), jnp.float32)], axis=1)
    a_rows = _sc_gather(table, dst.reshape(-1).astype(jnp.int32), E, _TD, 720)
    b_table = jnp.concatenate(
        [a_rows[:, 0:3], a_rows[:, 3 + H : 3 + 2 * H],
         jnp.zeros((E, _TDB - 3 - H), jnp.float32)], axis=1)
    b_rows = _sc_gather(b_table, perm_flat, E * SK, _TDB, 1800)

    ra = a_rows[:, 0:3].reshape(N * K, 1, 3)
    rb = b_rows[:, 0:3].reshape(N * K, SK, 3)
    diff = ra - rb + eps
    dist = jnp.sqrt(jnp.sum(diff * diff, axis=-1))

    g1 = a_rows[:, 3 : 3 + H].reshape(N * K, 1, H)
    g2 = b_rows[:, 3 : 3 + H].reshape(N * K, SK, H)
    gsum = g1 + g2 + b_gate
    gsum_t = jnp.transpose(gsum, (2, 0, 1))

    wbt_ext = jnp.concatenate(
        [W_bias.T, b_bias.reshape(H, 1)], axis=1)
    n_bias_blocks = (E * SK) // _BR
    bias_flat = pl.pallas_call(
        _bias_kernel,
        grid=(n_bias_blocks,),
        in_specs=[
            pl.BlockSpec((1, _BR), lambda i: (0, i)),
            pl.BlockSpec((H, _BR), lambda i: (0, i)),
            pl.BlockSpec((H, NUM_RBF + 1), lambda i: (0, 0)),
        ],
        out_specs=pl.BlockSpec((H, _BR), lambda i: (0, i)),
        out_shape=jax.ShapeDtypeStruct((H, E * SK), jnp.float32),
    )(dist.reshape(1, E * SK), gsum_t.reshape(H, E * SK), wbt_ext)
    bias_t = bias_flat.reshape(H, N * K, SK)

    wq = W_qk[:, :C_Z]
    wk = W_qk[:, C_Z:]
    bq = b_qk[:C_Z].reshape(1, C_Z)
    bk = b_qk[C_Z:].reshape(1, C_Z)
    bv = b_v.reshape(1, C_Z)
    bo = b_out.reshape(1, C_Z)

    grid = (N // BN,)
    out = pl.pallas_call(
        _main_kernel,
        grid=grid,
        in_specs=[
            pl.BlockSpec((BNK, C_Z), lambda i: (i, 0)),
            pl.BlockSpec((H, BNK, SK), lambda i: (0, i, 0)),
            pl.BlockSpec((C_Z, C_Z), lambda i: (0, 0)),
            pl.BlockSpec((C_Z, C_Z), lambda i: (0, 0)),
            pl.BlockSpec((C_Z, C_Z), lambda i: (0, 0)),
            pl.BlockSpec((C_Z, C_Z), lambda i: (0, 0)),
            pl.BlockSpec((1, C_Z), lambda i: (0, 0)),
            pl.BlockSpec((1, C_Z), lambda i: (0, 0)),
            pl.BlockSpec((1, C_Z), lambda i: (0, 0)),
            pl.BlockSpec((1, C_Z), lambda i: (0, 0)),
        ],
        out_specs=pl.BlockSpec((BNK, C_Z), lambda i: (i, 0)),
        out_shape=jax.ShapeDtypeStruct((E, C_Z), jnp.float32),
    )(edge_features, bias_t, wq, wk, W_v, W_out, bq, bk, bv, bo)
    return out


# speedup vs baseline: 10.0986x; 1.3640x over previous
import functools

import jax
import jax.numpy as jnp
import numpy as np
from jax import lax
from jax.experimental import pallas as pl
from jax.experimental.pallas import tpu as pltpu
from jax.experimental.pallas import tpu_sc as plsc

N = 384
K = 60
SK = 20
C_S = 384
C_Z = 128
H = 4
NUM_RBF = 64
DH = C_Z // H
E = N * K

BN = 8
BNK = BN * K
BSK = BN * SK

_D_MAX = 20.0
_MU_STEP = _D_MAX / (NUM_RBF - 1)
_INV_SIGMA = NUM_RBF / _D_MAX
_SCALE = 1.0 / float(np.sqrt(C_Z))


def _rotl32(x, d):
    return ((x << np.uint32(d)) | (x >> np.uint32(32 - d))).astype(np.uint32)


def _threefry2x32(k0, k1, x0, x1):
    x0 = x0.astype(np.uint32).copy()
    x1 = x1.astype(np.uint32).copy()
    k0 = np.uint32(k0)
    k1 = np.uint32(k1)
    k2 = np.uint32(k0 ^ k1 ^ np.uint32(0x1BD11BDA))
    rot = [(13, 15, 26, 6), (17, 29, 16, 24)]
    ks = [k0, k1, k2]
    x0 = (x0 + k0).astype(np.uint32)
    x1 = (x1 + k1).astype(np.uint32)
    for r in range(5):
        for d in rot[r % 2]:
            x0 = (x0 + x1).astype(np.uint32)
            x1 = _rotl32(x1, d)
            x1 = (x1 ^ x0).astype(np.uint32)
        x0 = (x0 + ks[(r + 1) % 3]).astype(np.uint32)
        x1 = (x1 + ks[(r + 2) % 3] + np.uint32(r + 1)).astype(np.uint32)
    return x0, x1


@functools.lru_cache(maxsize=1)
def _perm_flat_const():
    n = N * K * K
    b0, b1 = _threefry2x32(0, 42, np.zeros(n, np.uint32),
                           np.arange(n, dtype=np.uint32))
    bits = (b0 ^ b1).astype(np.uint32)
    fl = ((bits >> np.uint32(9)) | np.uint32(0x3F800000)).view(np.float32) \
        - np.float32(1.0)
    u = fl.reshape(N, K, K)
    idx_np = np.argsort(u, axis=-1, kind="stable")[..., :SK].astype(np.int32)
    flat = np.arange(N, dtype=np.int32)[:, None, None] * K + idx_np
    return flat.reshape(-1)


_TD = 16
_TDB = 8
_NW = 32


def _sc_gather(table, idx, rows, width, chunk):
    b_per_w = rows // _NW
    n_chunks = b_per_w // chunk
    mesh = plsc.VectorSubcoreMesh(core_axis_name="c", subcore_axis_name="s")

    @functools.partial(
        pl.kernel, mesh=mesh,
        out_type=jax.ShapeDtypeStruct((rows, width), jnp.float32),
        compiler_params=pltpu.CompilerParams(use_tc_tiling_on_sc=False),
        scratch_types=[
            pltpu.VMEM((chunk,), jnp.int32),
            pltpu.VMEM((chunk, width), jnp.float32),
            pltpu.SemaphoreType.DMA,
        ],
    )
    def k(table_hbm, idx_hbm, out_hbm, idx_v, rows_v, sem):
        wid = lax.axis_index("s") * 2 + lax.axis_index("c")
        base = wid * b_per_w

        def body(c, _):
            off = base + c * chunk
            pltpu.sync_copy(idx_hbm.at[pl.ds(off, chunk)], idx_v)
            pltpu.async_copy(table_hbm.at[idx_v], rows_v, sem).wait()
            pltpu.sync_copy(rows_v, out_hbm.at[pl.ds(off, chunk)])
            return ()

        lax.fori_loop(0, n_chunks, body, ())

    return k(table, idx)


_BP = 3840


@functools.lru_cache(maxsize=1)
def _rep_idx_const():
    return np.repeat(np.arange(E, dtype=np.int32), SK)


def _bias_kernel(a_ref, b_ref, wbt_ref, eps_ref, out_ref):
    i0 = jax.lax.broadcasted_iota(jnp.int32, (_TDB, _TDB), 0)
    i1 = jax.lax.broadcasted_iota(jnp.int32, (_TDB, _TDB), 1)
    ident = jnp.where(i0 == i1, 1.0, 0.0).astype(jnp.float32)
    at = jax.lax.dot_general(ident, a_ref[...], (((1,), (1,)), ((), ())),
                             preferred_element_type=jnp.float32)
    bt = jax.lax.dot_general(ident, b_ref[...], (((1,), (1,)), ((), ())),
                             preferred_element_type=jnp.float32)
    e = eps_ref[0, 0]
    x = at[0:1] - bt[0:1] + e
    y = at[1:2] - bt[1:2] + e
    z = at[2:3] - bt[2:3] + e
    dist = jnp.sqrt(x * x + y * y + z * z)
    mu = jax.lax.broadcasted_iota(jnp.int32, (NUM_RBF, _BP), 0)\
        .astype(jnp.float32) * _MU_STEP
    arg = (dist - mu) * _INV_SIGMA
    phi = jnp.exp(-(arg * arg))
    ones = jnp.ones((1, _BP), jnp.float32)
    phi_ext = jnp.concatenate([phi, ones], axis=0)
    raw = jnp.dot(wbt_ref[...], phi_ext,
                  preferred_element_type=jnp.float32)
    gs = at[3 : 3 + H] + bt[3 : 3 + H]
    out_ref[...] = jax.nn.sigmoid(gs) * raw


def _gate_proj_kernel(nf_ref, w_ref, out_ref):
    out_ref[...] = jnp.dot(nf_ref[...], w_ref[...],
                           preferred_element_type=jnp.float32)


def _main_kernel(edges_ref, bias_ref,
                 wq_ref, wk_ref, wv_ref, wo_ref,
                 bq_ref, bk_ref, bv_ref, bo_ref, out_ref):
    e = edges_ref[...]
    q = jnp.dot(e, wq_ref[...],
                preferred_element_type=jnp.float32) + bq_ref[...]
    e_sub = e.reshape(BN, K, C_Z)[:, :SK, :].reshape(BSK, C_Z)
    ks = jnp.dot(e_sub, wk_ref[...],
                 preferred_element_type=jnp.float32) + bk_ref[...]
    vs = jnp.dot(e_sub, wv_ref[...],
                 preferred_element_type=jnp.float32) + bv_ref[...]

    row_bn = jax.lax.broadcasted_iota(jnp.int32, (BNK, BSK), 0) // K
    col_bn = jax.lax.broadcasted_iota(jnp.int32, (BNK, BSK), 1) // SK
    mask = row_bn == col_bn

    u_parts = []
    for h in range(H):
        bias = bias_ref[h]
        qh = q[:, h * DH : (h + 1) * DH]
        kh = ks[:, h * DH : (h + 1) * DH]
        s = jax.lax.dot_general(qh, kh, (((1,), (1,)), ((), ())),
                                preferred_element_type=jnp.float32) * _SCALE
        bias_big = jnp.concatenate([bias] * BN, axis=1)
        s = jnp.where(mask, s + bias_big, -1e30)
        m = jnp.max(s, axis=1, keepdims=True)
        p = jnp.exp(s - m)
        p = p / jnp.sum(p, axis=1, keepdims=True)
        vh = vs[:, h * DH : (h + 1) * DH]
        u_parts.append(jnp.dot(p, vh, preferred_element_type=jnp.float32))
    u = jnp.concatenate(u_parts, axis=1)
    out_ref[...] = jnp.dot(u, wo_ref[...],
                           preferred_element_type=jnp.float32) + bo_ref[...]


def kernel(node_features, rigids, edge_features, edge_index,
           W_gate, b_gate, W_bias, b_bias, W_qk, b_qk, W_v, b_v,
           W_out, b_out, eps):
    dst = edge_index[0].reshape(N, K)
    perm_flat = jnp.asarray(_perm_flat_const())

    w_cat = jnp.concatenate([W_gate[:C_S], W_gate[C_S:]], axis=1)
    g = pl.pallas_call(
        _gate_proj_kernel,
        out_shape=jax.ShapeDtypeStruct((N, 2 * H), jnp.float32),
    )(node_features, w_cat)

    g_b = g + jnp.concatenate([b_gate, jnp.zeros((H,), jnp.float32)])
    table = jnp.concatenate(
        [rigids, g_b, jnp.zeros((N, _TD - 3 - 2 * H), jnp.float32)], axis=1)
    ab_edge = _sc_gather(table, dst.reshape(-1).astype(jnp.int32),
                         E, _TD, 720)
    pad1 = jnp.zeros((E, _TDB - 3 - H), jnp.float32)
    a_et = jnp.concatenate([ab_edge[:, 0:3], ab_edge[:, 3:3 + H], pad1], 1)
    b_et = jnp.concatenate(
        [ab_edge[:, 0:3], ab_edge[:, 3 + H : 3 + 2 * H], pad1], 1)
    a8 = _sc_gather(a_et, jnp.asarray(_rep_idx_const()), E * SK, _TDB, 1800)
    b8 = _sc_gather(b_et, perm_flat, E * SK, _TDB, 1800)

    wbt_ext = jnp.concatenate(
        [W_bias.T, b_bias.reshape(H, 1)], axis=1)
    n_bias_blocks = (E * SK) // _BP
    bias_flat = pl.pallas_call(
        _bias_kernel,
        grid=(n_bias_blocks,),
        in_specs=[
            pl.BlockSpec((_BP, _TDB), lambda i: (i, 0)),
            pl.BlockSpec((_BP, _TDB), lambda i: (i, 0)),
            pl.BlockSpec((H, NUM_RBF + 1), lambda i: (0, 0)),
            pl.BlockSpec((1, 1), lambda i: (0, 0)),
        ],
        out_specs=pl.BlockSpec((H, _BP), lambda i: (0, i)),
        out_shape=jax.ShapeDtypeStruct((H, E * SK), jnp.float32),
    )(a8, b8, wbt_ext, eps.reshape(1, 1))
    bias_t = bias_flat.reshape(H, N * K, SK)

    wq = W_qk[:, :C_Z]
    wk = W_qk[:, C_Z:]
    bq = b_qk[:C_Z].reshape(1, C_Z)
    bk = b_qk[C_Z:].reshape(1, C_Z)
    bv = b_v.reshape(1, C_Z)
    bo = b_out.reshape(1, C_Z)

    grid = (N // BN,)
    out = pl.pallas_call(
        _main_kernel,
        grid=grid,
        in_specs=[
            pl.BlockSpec((BNK, C_Z), lambda i: (i, 0)),
            pl.BlockSpec((H, BNK, SK), lambda i: (0, i, 0)),
            pl.BlockSpec((C_Z, C_Z), lambda i: (0, 0)),
            pl.BlockSpec((C_Z, C_Z), lambda i: (0, 0)),
            pl.BlockSpec((C_Z, C_Z), lambda i: (0, 0)),
            pl.BlockSpec((C_Z, C_Z), lambda i: (0, 0)),
            pl.BlockSpec((1, C_Z), lambda i: (0, 0)),
            pl.BlockSpec((1, C_Z), lambda i: (0, 0)),
            pl.BlockSpec((1, C_Z), lambda i: (0, 0)),
            pl.BlockSpec((1, C_Z), lambda i: (0, 0)),
        ],
        out_specs=pl.BlockSpec((BNK, C_Z), lambda i: (i, 0)),
        out_shape=jax.ShapeDtypeStruct((E, C_Z), jnp.float32),
    )(edge_features, bias_t, wq, wk, W_v, W_out, bq, bk, bv, bo)
    return out


# speedup vs baseline: 10.1099x; 1.0011x over previous
import functools

import jax
import jax.numpy as jnp
import numpy as np
from jax import lax
from jax.experimental import pallas as pl
from jax.experimental.pallas import tpu as pltpu
from jax.experimental.pallas import tpu_sc as plsc

N = 384
K = 60
SK = 20
C_S = 384
C_Z = 128
H = 4
NUM_RBF = 64
DH = C_Z // H
E = N * K

BN = 8
BNK = BN * K
BSK = BN * SK

_D_MAX = 20.0
_MU_STEP = _D_MAX / (NUM_RBF - 1)
_INV_SIGMA = NUM_RBF / _D_MAX
_SCALE = 1.0 / float(np.sqrt(C_Z))


def _rotl32(x, d):
    return ((x << np.uint32(d)) | (x >> np.uint32(32 - d))).astype(np.uint32)


def _threefry2x32(k0, k1, x0, x1):
    x0 = x0.astype(np.uint32).copy()
    x1 = x1.astype(np.uint32).copy()
    k0 = np.uint32(k0)
    k1 = np.uint32(k1)
    k2 = np.uint32(k0 ^ k1 ^ np.uint32(0x1BD11BDA))
    rot = [(13, 15, 26, 6), (17, 29, 16, 24)]
    ks = [k0, k1, k2]
    x0 = (x0 + k0).astype(np.uint32)
    x1 = (x1 + k1).astype(np.uint32)
    for r in range(5):
        for d in rot[r % 2]:
            x0 = (x0 + x1).astype(np.uint32)
            x1 = _rotl32(x1, d)
            x1 = (x1 ^ x0).astype(np.uint32)
        x0 = (x0 + ks[(r + 1) % 3]).astype(np.uint32)
        x1 = (x1 + ks[(r + 2) % 3] + np.uint32(r + 1)).astype(np.uint32)
    return x0, x1


@functools.lru_cache(maxsize=1)
def _perm_flat_const():
    n = N * K * K
    b0, b1 = _threefry2x32(0, 42, np.zeros(n, np.uint32),
                           np.arange(n, dtype=np.uint32))
    bits = (b0 ^ b1).astype(np.uint32)
    fl = ((bits >> np.uint32(9)) | np.uint32(0x3F800000)).view(np.float32) \
        - np.float32(1.0)
    u = fl.reshape(N, K, K)
    idx_np = np.argsort(u, axis=-1, kind="stable")[..., :SK].astype(np.int32)
    flat = np.arange(N, dtype=np.int32)[:, None, None] * K + idx_np
    return flat.reshape(-1)


_TD = 16
_TDB = 8
_NW = 32


def _sc_gather(table, idx, rows, width, chunk):
    b_per_w = rows // _NW
    n_chunks = b_per_w // chunk
    mesh = plsc.VectorSubcoreMesh(core_axis_name="c", subcore_axis_name="s")

    @functools.partial(
        pl.kernel, mesh=mesh,
        out_type=jax.ShapeDtypeStruct((rows, width), jnp.float32),
        compiler_params=pltpu.CompilerParams(use_tc_tiling_on_sc=False),
        scratch_types=[
            pltpu.VMEM((chunk,), jnp.int32),
            pltpu.VMEM((chunk, width), jnp.float32),
            pltpu.SemaphoreType.DMA,
        ],
    )
    def k(table_hbm, idx_hbm, out_hbm, idx_v, rows_v, sem):
        wid = lax.axis_index("s") * 2 + lax.axis_index("c")
        base = wid * b_per_w

        def body(c, _):
            off = base + c * chunk
            pltpu.sync_copy(idx_hbm.at[pl.ds(off, chunk)], idx_v)
            pltpu.async_copy(table_hbm.at[idx_v], rows_v, sem).wait()
            pltpu.sync_copy(rows_v, out_hbm.at[pl.ds(off, chunk)])
            return ()

        lax.fori_loop(0, n_chunks, body, ())

    return k(table, idx)


_BP = 3840


@functools.lru_cache(maxsize=1)
def _rep_idx_const():
    return np.repeat(np.arange(E, dtype=np.int32), SK)


def _bias_kernel(a_ref, b_ref, wbt_ref, eps_ref, out_ref):
    i0 = jax.lax.broadcasted_iota(jnp.int32, (_TDB, _TDB), 0)
    i1 = jax.lax.broadcasted_iota(jnp.int32, (_TDB, _TDB), 1)
    ident = jnp.where(i0 == i1, 1.0, 0.0).astype(jnp.float32)
    at = jax.lax.dot_general(ident, a_ref[...], (((1,), (1,)), ((), ())),
                             preferred_element_type=jnp.float32)
    bt = jax.lax.dot_general(ident, b_ref[...], (((1,), (1,)), ((), ())),
                             preferred_element_type=jnp.float32)
    e = eps_ref[0, 0]
    x = at[0:1] - bt[0:1] + e
    y = at[1:2] - bt[1:2] + e
    z = at[2:3] - bt[2:3] + e
    dist = jnp.sqrt(x * x + y * y + z * z)
    mu = jax.lax.broadcasted_iota(jnp.int32, (NUM_RBF, _BP), 0)\
        .astype(jnp.float32) * _MU_STEP
    arg = (dist - mu) * _INV_SIGMA
    phi = jnp.exp(-(arg * arg))
    ones = jnp.ones((1, _BP), jnp.float32)
    phi_ext = jnp.concatenate([phi, ones], axis=0)
    raw = jnp.dot(wbt_ref[...], phi_ext,
                  preferred_element_type=jnp.float32)
    gs = at[3 : 3 + H] + bt[3 : 3 + H]
    out_ref[...] = jax.nn.sigmoid(gs) * raw


def _gate_proj_kernel(nf_ref, w_ref, out_ref):
    out_ref[...] = jnp.dot(nf_ref[...], w_ref[...],
                           preferred_element_type=jnp.float32)


def _main_kernel(edges_ref, bias_ref,
                 wq_ref, wk_ref, wv_ref, wo_ref,
                 bq_ref, bk_ref, bv_ref, bo_ref, out_ref):
    e = edges_ref[...]
    q = jnp.dot(e, wq_ref[...],
                preferred_element_type=jnp.float32) + bq_ref[...]
    e_sub = e.reshape(BN, K, C_Z)[:, :SK, :].reshape(BSK, C_Z)
    ks = jnp.dot(e_sub, wk_ref[...],
                 preferred_element_type=jnp.float32) + bk_ref[...]
    vs = jnp.dot(e_sub, wv_ref[...],
                 preferred_element_type=jnp.float32) + bv_ref[...]

    row_bn = jax.lax.broadcasted_iota(jnp.int32, (BNK, BSK), 0) // K
    col_bn = jax.lax.broadcasted_iota(jnp.int32, (BNK, BSK), 1) // SK
    mask = row_bn == col_bn

    u_parts = []
    for h in range(H):
        bias = bias_ref[h]
        qh = q[:, h * DH : (h + 1) * DH]
        kh = ks[:, h * DH : (h + 1) * DH]
        s = jax.lax.dot_general(qh, kh, (((1,), (1,)), ((), ())),
                                preferred_element_type=jnp.float32) * _SCALE
        bias_big = jnp.concatenate([bias] * BN, axis=1)
        s = jnp.where(mask, s + bias_big, -1e30)
        m = jnp.max(s, axis=1, keepdims=True)
        p = jnp.exp(s - m)
        p = p / jnp.sum(p, axis=1, keepdims=True)
        vh = vs[:, h * DH : (h + 1) * DH]
        u_parts.append(jnp.dot(p, vh, preferred_element_type=jnp.float32))
    u = jnp.concatenate(u_parts, axis=1)
    out_ref[...] = jnp.dot(u, wo_ref[...],
                           preferred_element_type=jnp.float32) + bo_ref[...]


def kernel(node_features, rigids, edge_features, edge_index,
           W_gate, b_gate, W_bias, b_bias, W_qk, b_qk, W_v, b_v,
           W_out, b_out, eps):
    dst = edge_index[0].reshape(N, K)
    perm_flat = jnp.asarray(_perm_flat_const())

    w_cat = jnp.concatenate([W_gate[:C_S], W_gate[C_S:]], axis=1)
    g = pl.pallas_call(
        _gate_proj_kernel,
        out_shape=jax.ShapeDtypeStruct((N, 2 * H), jnp.float32),
    )(node_features, w_cat)

    g_b = g + jnp.concatenate([b_gate, jnp.zeros((H,), jnp.float32)])
    table = jnp.concatenate(
        [rigids, g_b, jnp.zeros((N, _TD - 3 - 2 * H), jnp.float32)], axis=1)
    ab_edge = _sc_gather(table, dst.reshape(-1).astype(jnp.int32),
                         E, _TD, 720)
    pad1 = jnp.zeros((E, _TDB - 3 - H), jnp.float32)
    a_et = jnp.concatenate([ab_edge[:, 0:3], ab_edge[:, 3:3 + H], pad1], 1)
    b_et = jnp.concatenate(
        [ab_edge[:, 0:3], ab_edge[:, 3 + H : 3 + 2 * H], pad1], 1)
    a8 = _sc_gather(a_et, jnp.asarray(_rep_idx_const()), E * SK, _TDB, 1800)
    b8 = _sc_gather(b_et, perm_flat, E * SK, _TDB, 1800)

    wbt_ext = jnp.concatenate(
        [W_bias.T, b_bias.reshape(H, 1)], axis=1)
    n_bias_blocks = (E * SK) // _BP
    bias_flat = pl.pallas_call(
        _bias_kernel,
        grid=(n_bias_blocks,),
        in_specs=[
            pl.BlockSpec((_BP, _TDB), lambda i: (i, 0)),
            pl.BlockSpec((_BP, _TDB), lambda i: (i, 0)),
            pl.BlockSpec((H, NUM_RBF + 1), lambda i: (0, 0)),
            pl.BlockSpec((1, 1), lambda i: (0, 0)),
        ],
        out_specs=pl.BlockSpec((H, _BP), lambda i: (0, i)),
        out_shape=jax.ShapeDtypeStruct((H, E * SK), jnp.float32),
    )(a8, b8, wbt_ext, jnp.asarray(eps, jnp.float32).reshape(1, 1))
    bias_t = bias_flat.reshape(H, N * K, SK)

    wq = W_qk[:, :C_Z]
    wk = W_qk[:, C_Z:]
    bq = b_qk[:C_Z].reshape(1, C_Z)
    bk = b_qk[C_Z:].reshape(1, C_Z)
    bv = b_v.reshape(1, C_Z)
    bo = b_out.reshape(1, C_Z)

    grid = (N // BN,)
    out = pl.pallas_call(
        _main_kernel,
        grid=grid,
        in_specs=[
            pl.BlockSpec((BNK, C_Z), lambda i: (i, 0)),
            pl.BlockSpec((H, BNK, SK), lambda i: (0, i, 0)),
            pl.BlockSpec((C_Z, C_Z), lambda i: (0, 0)),
            pl.BlockSpec((C_Z, C_Z), lambda i: (0, 0)),
            pl.BlockSpec((C_Z, C_Z), lambda i: (0, 0)),
            pl.BlockSpec((C_Z, C_Z), lambda i: (0, 0)),
            pl.BlockSpec((1, C_Z), lambda i: (0, 0)),
            pl.BlockSpec((1, C_Z), lambda i: (0, 0)),
            pl.BlockSpec((1, C_Z), lambda i: (0, 0)),
            pl.BlockSpec((1, C_Z), lambda i: (0, 0)),
        ],
        out_specs=pl.BlockSpec((BNK, C_Z), lambda i: (i, 0)),
        out_shape=jax.ShapeDtypeStruct((E, C_Z), jnp.float32),
    )(edge_features, bias_t, wq, wk, W_v, W_out, bq, bk, bv, bo)
    return out
